# Initial kernel scaffold; baseline (speedup 1.0000x reference)
import functools

import jax
import jax.numpy as jnp
from jax import lax
from jax.experimental import pallas as pl
from jax.experimental.pallas import tpu as pltpu
from jax.experimental.pallas import tpu_sc as plsc

VOCAB = 100000
DIM = 128
B_TOTAL = 4096 * 200
NUM_CORES = 2
NUM_SUBCORES = 16
NW = NUM_CORES * NUM_SUBCORES
PER_W = B_TOTAL // NW
CHUNK = 128
NCH = PER_W // CHUNK

_mesh = plsc.VectorSubcoreMesh(core_axis_name="c", subcore_axis_name="s")


@functools.partial(
    pl.kernel,
    mesh=_mesh,
    out_type=jax.ShapeDtypeStruct((B_TOTAL, DIM), jnp.float32),
    scratch_types=[
        pltpu.VMEM((NCH, CHUNK), jnp.int32),
        pltpu.VMEM((2, CHUNK, DIM), jnp.float32),
        pltpu.SemaphoreType.DMA,
        pltpu.SemaphoreType.DMA,
    ],
)
def _embed_sc(ids_hbm, table_hbm, out_hbm, idx_v, rows_v, gsem, osem):
    wid = lax.axis_index("s") * NUM_CORES + lax.axis_index("c")
    base = wid * PER_W

    pltpu.sync_copy(ids_hbm.at[pl.ds(wid * NCH, NCH)], idx_v)

    pltpu.async_copy(table_hbm.at[idx_v.at[0]], rows_v.at[0], gsem)

    def body(j, _):
        b = lax.rem(j, 2)
        pltpu.make_async_copy(table_hbm.at[idx_v.at[j]], rows_v.at[b], gsem).wait()

        @pl.when(j + 1 < NCH)
        def _():
            pltpu.async_copy(table_hbm.at[idx_v.at[j + 1]], rows_v.at[1 - b], gsem)

        @pl.when(j >= 2)
        def _():
            pltpu.make_async_copy(
                rows_v.at[b], out_hbm.at[pl.ds(base + (j - 2) * CHUNK, CHUNK)], osem
            ).wait()

        pltpu.async_copy(
            rows_v.at[b], out_hbm.at[pl.ds(base + j * CHUNK, CHUNK)], osem
        )
        return 0

    lax.fori_loop(0, NCH, body, 0)

    pltpu.make_async_copy(
        rows_v.at[NCH % 2], out_hbm.at[pl.ds(base + (NCH - 2) * CHUNK, CHUNK)], osem
    ).wait()
    pltpu.make_async_copy(
        rows_v.at[1 - (NCH % 2)],
        out_hbm.at[pl.ds(base + (NCH - 1) * CHUNK, CHUNK)],
        osem,
    ).wait()


def kernel(input_ids, weight):
    ids2d = input_ids.reshape(B_TOTAL // CHUNK, CHUNK)
    out = _embed_sc(ids2d, weight)
    return out.reshape(input_ids.shape[0], input_ids.shape[1], DIM)


# speedup vs baseline: 7.5656x; 7.5656x over previous
import functools

import jax
import jax.numpy as jnp
from jax import lax
from jax.experimental import pallas as pl
from jax.experimental.pallas import tpu as pltpu
from jax.experimental.pallas import tpu_sc as plsc

VOCAB = 100000
DIM = 128
B_TOTAL = 4096 * 200
NUM_CORES = 2
NUM_SUBCORES = 16
NW = NUM_CORES * NUM_SUBCORES
PER_W = B_TOTAL // NW
CHUNK = 128
NCH = PER_W // CHUNK

_mesh = plsc.VectorSubcoreMesh(core_axis_name="c", subcore_axis_name="s")


@functools.partial(
    pl.kernel,
    mesh=_mesh,
    out_type=jax.ShapeDtypeStruct((B_TOTAL, DIM), jnp.float32),
    scratch_types=[
        pltpu.VMEM((NCH, CHUNK), jnp.int32),
        pltpu.VMEM((2, CHUNK, DIM), jnp.float32),
        pltpu.SemaphoreType.DMA,
        pltpu.SemaphoreType.DMA,
    ],
)
def _embed_sc(ids_hbm, table_hbm, out_hbm, idx_v, rows_v, gsem, osem):
    wid = lax.axis_index("s") * NUM_CORES + lax.axis_index("c")
    base = wid * PER_W

    pltpu.sync_copy(ids_hbm.at[pl.ds(wid * NCH, NCH)], idx_v)

    pltpu.async_copy(table_hbm.at[idx_v.at[0]], rows_v.at[0], gsem)

    def body(j, _):
        b = lax.rem(j, 2)
        pltpu.make_async_copy(table_hbm.at[idx_v.at[j]], rows_v.at[b], gsem).wait()

        @pl.when(j >= 1)
        def _():
            pltpu.make_async_copy(
                rows_v.at[1 - b], out_hbm.at[pl.ds(base + (j - 1) * CHUNK, CHUNK)], osem
            ).wait()

        @pl.when(j + 1 < NCH)
        def _():
            pltpu.async_copy(table_hbm.at[idx_v.at[j + 1]], rows_v.at[1 - b], gsem)

        pltpu.async_copy(
            rows_v.at[b], out_hbm.at[pl.ds(base + j * CHUNK, CHUNK)], osem
        )
        return 0

    lax.fori_loop(0, NCH, body, 0)

    pltpu.make_async_copy(
        rows_v.at[(NCH - 1) % 2],
        out_hbm.at[pl.ds(base + (NCH - 1) * CHUNK, CHUNK)],
        osem,
    ).wait()


def kernel(input_ids, weight):
    ids2d = input_ids.reshape(B_TOTAL // CHUNK, CHUNK)
    out = _embed_sc(ids2d, weight)
    return out.reshape(input_ids.shape[0], input_ids.shape[1], DIM)


# speedup vs baseline: 9.2951x; 1.2286x over previous
import functools

import jax
import jax.numpy as jnp
from jax import lax
from jax.experimental import pallas as pl
from jax.experimental.pallas import tpu as pltpu
from jax.experimental.pallas import tpu_sc as plsc

VOCAB = 100000
DIM = 128
B_TOTAL = 4096 * 200
NUM_CORES = 2
NUM_SUBCORES = 16
NW = NUM_CORES * NUM_SUBCORES
PER_W = B_TOTAL // NW
CHUNK = 128
NBUF = 4
NCH = PER_W // CHUNK

_mesh = plsc.VectorSubcoreMesh(core_axis_name="c", subcore_axis_name="s")


@functools.partial(
    pl.kernel,
    mesh=_mesh,
    out_type=jax.ShapeDtypeStruct((B_TOTAL, DIM), jnp.float32),
    scratch_types=[
        pltpu.VMEM((NCH, CHUNK), jnp.int32),
        pltpu.VMEM((NBUF, CHUNK, DIM), jnp.float32),
        pltpu.SemaphoreType.DMA,
        pltpu.SemaphoreType.DMA,
    ],
)
def _embed_sc(ids_hbm, table_hbm, out_hbm, idx_v, rows_v, gsem, osem):
    wid = lax.axis_index("s") * NUM_CORES + lax.axis_index("c")
    base = wid * PER_W

    pltpu.sync_copy(ids_hbm.at[wid], idx_v)

    pltpu.async_copy(table_hbm.at[idx_v.at[0]], rows_v.at[0], gsem)
    pltpu.async_copy(table_hbm.at[idx_v.at[1]], rows_v.at[1], gsem)

    def body(j, _):
        b = lax.rem(j, NBUF)
        pltpu.make_async_copy(table_hbm.at[idx_v.at[j]], rows_v.at[b], gsem).wait()

        @pl.when(j >= 2)
        def _():
            pltpu.make_async_copy(
                rows_v.at[lax.rem(j + 2, NBUF)],
                out_hbm.at[pl.ds(base + (j - 2) * CHUNK, CHUNK)],
                osem,
            ).wait()

        @pl.when(j + 2 < NCH)
        def _():
            pltpu.async_copy(
                table_hbm.at[idx_v.at[j + 2]], rows_v.at[lax.rem(j + 2, NBUF)], gsem
            )

        pltpu.async_copy(
            rows_v.at[b], out_hbm.at[pl.ds(base + j * CHUNK, CHUNK)], osem
        )
        return 0

    lax.fori_loop(0, NCH, body, 0)

    pltpu.make_async_copy(
        rows_v.at[(NCH - 2) % NBUF],
        out_hbm.at[pl.ds(base + (NCH - 2) * CHUNK, CHUNK)],
        osem,
    ).wait()
    pltpu.make_async_copy(
        rows_v.at[(NCH - 1) % NBUF],
        out_hbm.at[pl.ds(base + (NCH - 1) * CHUNK, CHUNK)],
        osem,
    ).wait()


def kernel(input_ids, weight):
    ids3d = input_ids.reshape(NW, NCH, CHUNK)
    out = _embed_sc(ids3d, weight)
    return out.reshape(input_ids.shape[0], input_ids.shape[1], DIM)


# speedup vs baseline: 9.3549x; 1.0064x over previous
import functools

import jax
import jax.numpy as jnp
from jax import lax
from jax.experimental import pallas as pl
from jax.experimental.pallas import tpu as pltpu
from jax.experimental.pallas import tpu_sc as plsc

VOCAB = 100000
DIM = 128
B_TOTAL = 4096 * 200
NUM_CORES = 2
NUM_SUBCORES = 16
NW = NUM_CORES * NUM_SUBCORES
PER_W = B_TOTAL // NW
CHUNK = 128
NBUF = 6
DEPTH = 3
NCH = PER_W // CHUNK

_mesh = plsc.VectorSubcoreMesh(core_axis_name="c", subcore_axis_name="s")


@functools.partial(
    pl.kernel,
    mesh=_mesh,
    out_type=jax.ShapeDtypeStruct((B_TOTAL, DIM), jnp.float32),
    scratch_types=[
        pltpu.VMEM((NCH, CHUNK), jnp.int32),
        pltpu.VMEM((NBUF, CHUNK, DIM), jnp.float32),
        pltpu.SemaphoreType.DMA,
        pltpu.SemaphoreType.DMA,
    ],
)
def _embed_sc(ids_hbm, table_hbm, out_hbm, idx_v, rows_v, gsem, osem):
    wid = lax.axis_index("s") * NUM_CORES + lax.axis_index("c")
    base = wid * PER_W

    pltpu.sync_copy(ids_hbm.at[wid], idx_v)

    for p in range(DEPTH):
        pltpu.async_copy(table_hbm.at[idx_v.at[p]], rows_v.at[p], gsem)

    lag = NBUF - DEPTH

    def body(j, _):
        b = lax.rem(j, NBUF)
        pltpu.make_async_copy(table_hbm.at[idx_v.at[j]], rows_v.at[b], gsem).wait()

        @pl.when(j >= lag)
        def _():
            pltpu.make_async_copy(
                rows_v.at[lax.rem(j + DEPTH, NBUF)],
                out_hbm.at[pl.ds(base + (j - lag) * CHUNK, CHUNK)],
                osem,
            ).wait()

        @pl.when(j + DEPTH < NCH)
        def _():
            pltpu.async_copy(
                table_hbm.at[idx_v.at[j + DEPTH]],
                rows_v.at[lax.rem(j + DEPTH, NBUF)],
                gsem,
            )

        pltpu.async_copy(
            rows_v.at[b], out_hbm.at[pl.ds(base + j * CHUNK, CHUNK)], osem
        )
        return 0

    lax.fori_loop(0, NCH, body, 0)

    for p in range(NCH - lag, NCH):
        pltpu.make_async_copy(
            rows_v.at[p % NBUF],
            out_hbm.at[pl.ds(base + p * CHUNK, CHUNK)],
            osem,
        ).wait()


def kernel(input_ids, weight):
    ids3d = input_ids.reshape(NW, NCH, CHUNK)
    out = _embed_sc(ids3d, weight)
    return out.reshape(input_ids.shape[0], input_ids.shape[1], DIM)
